# baseline (speedup 1.0000x reference)
import jax
import jax.numpy as jnp
from jax.experimental import pallas as pl

_B, _N, _DIN, _DHID, _DOUT = 32, 2048, 32, 16, 64
_EPS = 1e-5


def _body(xt_ref, adj_ref, w1b_ref, bias1_ref, w2b_ref, scale_ref,
          shift_ref, bias2_ref, fcwp_ref, fcb_ref, out_ref):
    f32 = jnp.float32
    xw = jnp.dot(xt_ref[...], w1b_ref[...], preferred_element_type=f32)
    adj = adj_ref[...]
    h1 = jnp.dot(adj, xw, preferred_element_type=f32) + bias1_ref[...]
    mean = jnp.mean(h1, axis=1, keepdims=True)
    var = jnp.mean(h1 * h1, axis=1, keepdims=True) - mean * mean
    s = scale_ref[...] * jax.lax.rsqrt(var + _EPS)
    t = shift_ref[...] - mean * s
    bnh1 = h1 * s + t
    s2 = jnp.dot(bnh1, w2b_ref[...], preferred_element_type=f32)
    h2 = jnp.dot(adj, s2, preferred_element_type=f32) + bias2_ref[...]
    acc = jnp.zeros((_DOUT, _B), dtype=f32)
    for h in range(_DHID):
        blk = h2[:, _B * h:_B * (h + 1)]
        acc = acc + jnp.dot(fcwp_ref[h], blk,
                            preferred_element_type=f32)
    out_ref[...] = acc.T + fcb_ref[...]


def kernel(x, network, W1, b1, W2, b2, gamma, beta, fcW, fcb):
    f32 = jnp.float32
    xt = jnp.transpose(x, (1, 0, 2)).reshape(_N, _B * _DIN)
    eye = jnp.eye(_B, dtype=f32)
    w1b = (eye[:, None, None, :] * W1[None, :, :, None]).reshape(
        _B * _DIN, _DHID * _B)
    w2b = jnp.kron(W2, eye)
    bias1 = jnp.repeat(b1, _B)[None, :]
    bias2 = jnp.repeat(b2, _B)[None, :]
    fcwp = fcW.reshape(_DOUT, _N, _DHID).transpose(2, 0, 1)
    return pl.pallas_call(
        _body,
        out_shape=jax.ShapeDtypeStruct((_B, _DOUT), f32),
    )(xt, network, w1b, bias1, w2b, gamma[:, None], beta[:, None],
      bias2, fcwp, fcb[None, :])


# speedup vs baseline: 1.0392x; 1.0392x over previous
import jax
import jax.numpy as jnp
from jax.experimental import pallas as pl

_B, _N, _DIN, _DHID, _DOUT = 32, 2048, 32, 16, 64
_EPS = 1e-5


def _body(xt_ref, adj_ref, w1b_ref, bias1_ref, w2b_ref, scale_ref,
          shift_ref, bias2_ref, fcwp_ref, fcb_ref, out_ref):
    f32 = jnp.float32
    bf16 = jnp.bfloat16
    xw = jnp.dot(xt_ref[...].astype(bf16), w1b_ref[...].astype(bf16),
                 preferred_element_type=f32)
    adj = adj_ref[...].astype(bf16)
    h1 = jnp.dot(adj, xw.astype(bf16),
                 preferred_element_type=f32) + bias1_ref[...]
    mean = jnp.mean(h1, axis=1, keepdims=True)
    var = jnp.mean(h1 * h1, axis=1, keepdims=True) - mean * mean
    s = scale_ref[...] * jax.lax.rsqrt(var + _EPS)
    t = shift_ref[...] - mean * s
    bnh1 = h1 * s + t
    s2 = jnp.dot(bnh1.astype(bf16), w2b_ref[...].astype(bf16),
                 preferred_element_type=f32)
    h2 = jnp.dot(adj, s2.astype(bf16),
                 preferred_element_type=f32) + bias2_ref[...]
    acc = jnp.zeros((_DOUT, _B), dtype=f32)
    for h in range(_DHID):
        blk = h2[:, _B * h:_B * (h + 1)]
        acc = acc + jnp.dot(fcwp_ref[h].astype(bf16), blk.astype(bf16),
                            preferred_element_type=f32)
    out_ref[...] = acc.T + fcb_ref[...]


def kernel(x, network, W1, b1, W2, b2, gamma, beta, fcW, fcb):
    f32 = jnp.float32
    xt = jnp.transpose(x, (1, 0, 2)).reshape(_N, _B * _DIN)
    eye = jnp.eye(_B, dtype=f32)
    w1b = (eye[:, None, None, :] * W1[None, :, :, None]).reshape(
        _B * _DIN, _DHID * _B)
    w2b = jnp.kron(W2, eye)
    bias1 = jnp.repeat(b1, _B)[None, :]
    bias2 = jnp.repeat(b2, _B)[None, :]
    fcwp = fcW.reshape(_DOUT, _N, _DHID).transpose(2, 0, 1)
    return pl.pallas_call(
        _body,
        out_shape=jax.ShapeDtypeStruct((_B, _DOUT), f32),
    )(xt, network, w1b, bias1, w2b, gamma[:, None], beta[:, None],
      bias2, fcwp, fcb[None, :])


# speedup vs baseline: 1.1640x; 1.1201x over previous
import jax
import jax.numpy as jnp
from jax.experimental import pallas as pl
from jax.experimental.pallas import tpu as pltpu

_B, _N, _DIN, _DHID, _DOUT = 32, 2048, 32, 16, 64
_EPS = 1e-5
_BLK = 256
_NBLK = _N // _BLK
_HB = _DHID * _B


def _body(xt_ref, w1b_ref, adj_ref, w2b_ref, scale_ref, shift_ref,
          bias1_ref, bias2_ref, fcwp_ref, fcb_ref, out_ref,
          xw_ref, s2_ref, acc_ref):
    f32 = jnp.float32
    bf16 = jnp.bfloat16
    i = pl.program_id(0)

    @pl.when(i == 0)
    def _prep():
        xw_ref[...] = jnp.dot(
            xt_ref[...], w1b_ref[...],
            preferred_element_type=f32).astype(bf16)

    @pl.when(jnp.logical_and(i >= 1, i <= _NBLK))
    def _hop1():
        j = i - 1
        adj = adj_ref[...].astype(bf16)
        h1 = jnp.dot(adj, xw_ref[...],
                     preferred_element_type=f32) + bias1_ref[...]
        mean = jnp.mean(h1, axis=1, keepdims=True)
        var = jnp.mean(h1 * h1, axis=1, keepdims=True) - mean * mean
        sc = scale_ref[pl.ds(j * _BLK, _BLK), :] * jax.lax.rsqrt(var + _EPS)
        t = shift_ref[pl.ds(j * _BLK, _BLK), :] - mean * sc
        bnh1 = h1 * sc + t
        s2_ref[pl.ds(j * _BLK, _BLK), :] = jnp.dot(
            bnh1.astype(bf16), w2b_ref[...],
            preferred_element_type=f32).astype(bf16)

    @pl.when(i > _NBLK)
    def _hop2():
        adj = adj_ref[...].astype(bf16)
        h2 = jnp.dot(adj, s2_ref[...],
                     preferred_element_type=f32) + bias2_ref[...]
        part = jnp.zeros((_DOUT, _B), dtype=f32)
        for h in range(_DHID):
            blk = h2[:, _B * h:_B * (h + 1)].astype(bf16)
            part = part + jnp.dot(fcwp_ref[h], blk,
                                  preferred_element_type=f32)

        @pl.when(i == _NBLK + 1)
        def _():
            acc_ref[...] = part

        @pl.when(i > _NBLK + 1)
        def _():
            acc_ref[...] = acc_ref[...] + part

        @pl.when(i == 2 * _NBLK)
        def _():
            out_ref[...] = acc_ref[...].T + fcb_ref[...]


def _adj_index(i):
    return (jnp.where(i <= _NBLK, jnp.clip(i - 1, 0, _NBLK - 1), i - 1 - _NBLK), 0)


def kernel(x, network, W1, b1, W2, b2, gamma, beta, fcW, fcb):
    f32 = jnp.float32
    bf16 = jnp.bfloat16
    xt = jnp.transpose(x, (1, 0, 2)).reshape(_N, _B * _DIN).astype(bf16)
    eye = jnp.eye(_B, dtype=f32)
    w1b = (eye[:, None, None, :] * W1[None, :, :, None]).reshape(
        _B * _DIN, _HB).astype(bf16)
    w2b = jnp.kron(W2, eye).astype(bf16)
    bias1 = jnp.repeat(b1, _B)[None, :]
    bias2 = jnp.repeat(b2, _B)[None, :]
    fcwp = fcW.reshape(_DOUT, _N, _DHID).transpose(2, 0, 1).astype(bf16)

    grid = (2 * _NBLK + 1,)
    return pl.pallas_call(
        _body,
        grid=grid,
        in_specs=[
            pl.BlockSpec((_N, _B * _DIN), lambda i: (0, 0)),
            pl.BlockSpec((_B * _DIN, _HB), lambda i: (0, 0)),
            pl.BlockSpec((_BLK, _N), _adj_index),
            pl.BlockSpec((_HB, _HB), lambda i: (0, 0)),
            pl.BlockSpec((_N, 1), lambda i: (0, 0)),
            pl.BlockSpec((_N, 1), lambda i: (0, 0)),
            pl.BlockSpec((1, _HB), lambda i: (0, 0)),
            pl.BlockSpec((1, _HB), lambda i: (0, 0)),
            pl.BlockSpec((_DHID, _DOUT, _BLK),
                         lambda i: (0, 0, jnp.clip(i - 1 - _NBLK, 0,
                                                   _NBLK - 1))),
            pl.BlockSpec((1, _DOUT), lambda i: (0, 0)),
        ],
        out_specs=pl.BlockSpec((_B, _DOUT), lambda i: (0, 0)),
        out_shape=jax.ShapeDtypeStruct((_B, _DOUT), f32),
        scratch_shapes=[
            pltpu.VMEM((_N, _HB), bf16),
            pltpu.VMEM((_N, _HB), bf16),
            pltpu.VMEM((_DOUT, _B), f32),
        ],
        compiler_params=pltpu.CompilerParams(
            dimension_semantics=("arbitrary",)),
    )(xt, w1b, network, w2b, gamma[:, None], beta[:, None],
      bias1, bias2, fcwp, fcb[None, :])
